# baseline (speedup 1.0000x reference)
import functools

import jax
import jax.numpy as jnp
from jax import lax
from jax.experimental import pallas as pl
from jax.experimental.pallas import tpu as pltpu
from jax.experimental.pallas import tpu_sc as plsc

NUM_CORES = 2
NUM_SUBCORES = 16
NUM_WORKERS = NUM_CORES * NUM_SUBCORES
SUBLANE = 8
LANES = 16

_COMPILER_PARAMS = pltpu.CompilerParams(
    use_tc_tiling_on_sc=False, needs_layout_passes=False
)


def _diag_vectors():
    iota = lax.iota(jnp.int32, LANES)
    rots = [jnp.bitwise_and(iota + k, LANES - 1) for k in range(LANES)]
    return iota, rots


def _transpose_tile(src, dst, rows, cols, iota, rots, loop_rows=True):
    def block(r0, c0):
        for k in range(LANES):
            r_ids = r0 * LANES + iota
            c_ids = c0 * LANES + rots[k]
            vals = plsc.load_gather(src, [r_ids, c_ids])
            plsc.store_scatter(dst, [c_ids, r_ids], vals)

    if loop_rows:
        def r_body(r0, carry):
            for c0 in range(cols // LANES):
                block(r0, c0)
            return carry

        lax.fori_loop(0, rows // LANES, r_body, 0)
    else:
        def c_body(c0, carry):
            for r0 in range(rows // LANES):
                block(r0, c0)
            return carry

        lax.fori_loop(0, cols // LANES, c_body, 0)


@functools.lru_cache(maxsize=None)
def _make_transpose(V, D):
    CH = 256
    n_full = V // CH
    tail = V - n_full * CH
    n_rr = n_full // NUM_WORKERS
    n_extra = n_full - n_rr * NUM_WORKERS
    assert n_rr % 2 == 0 and tail % (2 * LANES) == 0
    mesh = plsc.VectorSubcoreMesh(core_axis_name="c", subcore_axis_name="s")

    @functools.partial(
        pl.kernel,
        mesh=mesh,
        out_type=jax.ShapeDtypeStruct((V, D), jnp.float32),
        scratch_types=[
            pltpu.VMEM((2, D, CH), jnp.float32),
            pltpu.VMEM((2, CH, D), jnp.float32),
            pltpu.VMEM((D, tail), jnp.float32),
            pltpu.VMEM((tail, D), jnp.float32),
            pltpu.SemaphoreType.DMA((2,)),
            pltpu.SemaphoreType.DMA((2,)),
        ],
        compiler_params=_COMPILER_PARAMS,
    )
    def tkern(tabT_hbm, out_hbm, in_v, tr_v, tin_v, ttr_v, sem_r, sem_w):
        wid = lax.axis_index("s") * NUM_CORES + lax.axis_index("c")
        iota, rots = _diag_vectors()

        def col0_of(t):
            return (wid + NUM_WORKERS * t) * CH

        def rd_start(t, b):
            pltpu.async_copy(
                tabT_hbm.at[:, pl.ds(col0_of(t), CH)], in_v.at[b], sem_r.at[b]
            )

        def rd_wait(b):
            pltpu.make_async_copy(
                tabT_hbm.at[:, pl.ds(0, CH)], in_v.at[b], sem_r.at[b]
            ).wait()

        def wr_start(t, b):
            pltpu.async_copy(
                tr_v.at[b], out_hbm.at[pl.ds(col0_of(t), CH), :], sem_w.at[b]
            )

        def wr_wait(b):
            pltpu.make_async_copy(
                tr_v.at[b], out_hbm.at[pl.ds(0, CH), :], sem_w.at[b]
            ).wait()

        for b in range(2):
            rd_start(b, b)

        def super_body(s, carry):
            for j in range(2):
                t = s * 2 + j
                b = j

                @pl.when(s > 0)
                def _():
                    wr_wait(b)

                rd_wait(b)
                _transpose_tile(
                    in_v.at[b], tr_v.at[b], D, CH, iota, rots, loop_rows=False
                )

                @pl.when(s < n_rr // 2 - 1)
                def _():
                    rd_start(t + 2, b)

                wr_start(t, b)
            return carry

        lax.fori_loop(0, n_rr // 2, super_body, 0)

        for b in range(2):
            wr_wait(b)

        for e in range(n_extra):
            @pl.when(wid == e)
            def _():
                col0 = (n_rr * NUM_WORKERS + e) * CH
                pltpu.sync_copy(tabT_hbm.at[:, pl.ds(col0, CH)], in_v.at[0])
                _transpose_tile(
                    in_v.at[0], tr_v.at[0], D, CH, iota, rots, loop_rows=False
                )
                pltpu.sync_copy(tr_v.at[0], out_hbm.at[pl.ds(col0, CH), :])

        if tail:
            @pl.when(wid == n_extra)
            def _():
                col0 = n_full * CH
                pltpu.sync_copy(tabT_hbm.at[:, pl.ds(col0, tail)], tin_v)
                _transpose_tile(
                    tin_v, ttr_v, D, tail, iota, rots, loop_rows=False
                )
                pltpu.sync_copy(ttr_v, out_hbm.at[pl.ds(col0, tail), :])

    return tkern


@functools.lru_cache(maxsize=None)
def _make_lookup(V, D, Bt, H):
    BLK = Bt // NUM_WORKERS
    assert Bt % NUM_WORKERS == 0 and BLK % LANES == 0 and D % LANES == 0
    assert H % 2 == 0
    mesh = plsc.VectorSubcoreMesh(core_axis_name="c", subcore_axis_name="s")

    @functools.partial(
        pl.kernel,
        mesh=mesh,
        out_type=jax.ShapeDtypeStruct((H, D, Bt), jnp.float32),
        scratch_types=[
            pltpu.VMEM((2, BLK), jnp.int32),
            pltpu.VMEM((2, BLK, D), jnp.float32),
            pltpu.VMEM((2, D, BLK), jnp.float32),
            pltpu.SemaphoreType.DMA((2,)),
            pltpu.SemaphoreType.DMA((2,)),
            pltpu.SemaphoreType.DMA((2,)),
        ],
        compiler_params=_COMPILER_PARAMS,
    )
    def lookup(tab_hbm, idx_hbm, out_hbm, idx_v, rows_v, tr_v, sem_i, sem_g, sem_o):
        wid = lax.axis_index("s") * NUM_CORES + lax.axis_index("c")
        col0 = wid * BLK
        iota, rots = _diag_vectors()

        def idx_off(t):
            return t * Bt + col0

        def idx_start(t, b):
            pltpu.async_copy(
                idx_hbm.at[pl.ds(idx_off(t), BLK)], idx_v.at[b], sem_i.at[b]
            )

        def idx_wait(b):
            pltpu.make_async_copy(
                idx_hbm.at[pl.ds(0, BLK)], idx_v.at[b], sem_i.at[b]
            ).wait()

        def gather_start(b):
            pltpu.async_copy(tab_hbm.at[idx_v.at[b]], rows_v.at[b], sem_g.at[b])

        def gather_wait(b):
            pltpu.make_async_copy(
                tab_hbm.at[idx_v.at[b]], rows_v.at[b], sem_g.at[b]
            ).wait()

        def write_start(t, b):
            pltpu.async_copy(
                tr_v.at[b], out_hbm.at[t, :, pl.ds(col0, BLK)], sem_o.at[b]
            )

        def write_wait(b):
            pltpu.make_async_copy(
                tr_v.at[b], out_hbm.at[0, :, pl.ds(0, BLK)], sem_o.at[b]
            ).wait()

        for b in range(2):
            pltpu.sync_copy(idx_hbm.at[pl.ds(idx_off(b), BLK)], idx_v.at[b])
            gather_start(b)

        def super_body(s, carry):
            for j in range(2):
                t = s * 2 + j
                b = j

                @pl.when(s > 0)
                def _():
                    write_wait(b)

                gather_wait(b)

                @pl.when(s < (H // 2) - 1)
                def _():
                    idx_start(t + 2, b)

                _transpose_tile(rows_v.at[b], tr_v.at[b], BLK, D, iota, rots)
                write_start(t, b)

                @pl.when(s < (H // 2) - 1)
                def _():
                    idx_wait(b)
                    gather_start(b)

            return carry

        lax.fori_loop(0, H // 2, super_body, 0)

        for b in range(2):
            write_wait(b)

    return lookup


def kernel(input_seq, embedding_table):
    Bt, H = input_seq.shape
    V, D = embedding_table.shape
    Hp = (H + SUBLANE - 1) // SUBLANE * SUBLANE
    tab_rm = _make_transpose(V, D)(embedding_table.T)
    idxT = jnp.pad(input_seq.T.astype(jnp.int32), ((0, Hp - H), (0, 0)))
    outT = _make_lookup(V, D, Bt, H)(tab_rm, idxT.reshape(Hp * Bt))
    return outT.transpose(2, 0, 1)


# speedup vs baseline: 4.3727x; 4.3727x over previous
import functools

import jax
import jax.numpy as jnp
from jax import lax
from jax.experimental import pallas as pl
from jax.experimental.pallas import tpu as pltpu
from jax.experimental.pallas import tpu_sc as plsc

NUM_CORES = 2
NUM_SUBCORES = 16
NUM_WORKERS = NUM_CORES * NUM_SUBCORES
SUBLANE = 8
LANES = 16


@functools.lru_cache(maxsize=None)
def _make_lookup(V, D, Bt, H):
    BLK = Bt // NUM_WORKERS
    assert Bt % NUM_WORKERS == 0 and BLK % LANES == 0 and D % LANES == 0
    assert H % 2 == 0
    mesh = plsc.VectorSubcoreMesh(core_axis_name="c", subcore_axis_name="s")

    @functools.partial(
        pl.kernel,
        mesh=mesh,
        out_type=jax.ShapeDtypeStruct((H, D, Bt), jnp.float32),
        scratch_types=[
            pltpu.VMEM((2, BLK), jnp.int32),
            pltpu.VMEM((2, BLK, D), jnp.float32),
            pltpu.VMEM((2, D, BLK), jnp.float32),
            pltpu.SemaphoreType.DMA((2,)),
            pltpu.SemaphoreType.DMA((2,)),
            pltpu.SemaphoreType.DMA((2,)),
        ],
        compiler_params=pltpu.CompilerParams(
            use_tc_tiling_on_sc=False, needs_layout_passes=False
        ),
    )
    def lookup(tab_hbm, idx_hbm, out_hbm, idx_v, rows_v, tr_v, sem_i, sem_g, sem_o):
        wid = lax.axis_index("s") * NUM_CORES + lax.axis_index("c")
        col0 = wid * BLK
        iota = lax.iota(jnp.int32, LANES)
        rots = [jnp.bitwise_and(iota + k, LANES - 1) for k in range(LANES)]

        def idx_off(t):
            return t * Bt + col0

        def idx_start(t, b):
            pltpu.async_copy(
                idx_hbm.at[pl.ds(idx_off(t), BLK)], idx_v.at[b], sem_i.at[b]
            )

        def idx_wait(b):
            pltpu.make_async_copy(
                idx_hbm.at[pl.ds(0, BLK)], idx_v.at[b], sem_i.at[b]
            ).wait()

        def gather_start(b):
            pltpu.async_copy(tab_hbm.at[idx_v.at[b]], rows_v.at[b], sem_g.at[b])

        def gather_wait(b):
            pltpu.make_async_copy(
                tab_hbm.at[idx_v.at[b]], rows_v.at[b], sem_g.at[b]
            ).wait()

        def write_start(t, b):
            pltpu.async_copy(
                tr_v.at[b], out_hbm.at[t, :, pl.ds(col0, BLK)], sem_o.at[b]
            )

        def write_wait(b):
            pltpu.make_async_copy(
                tr_v.at[b], out_hbm.at[0, :, pl.ds(0, BLK)], sem_o.at[b]
            ).wait()

        def transpose(b):
            @plsc.parallel_loop(0, BLK // LANES, 1, unroll=2)
            def r_body(r0):
                row_ids = r0 * LANES + iota
                for c0 in range(D // LANES):
                    for k in range(LANES):
                        col_ids = c0 * LANES + rots[k]
                        vals = plsc.load_gather(
                            rows_v.at[b], [row_ids, col_ids]
                        )
                        plsc.store_scatter(
                            tr_v.at[b], [col_ids, row_ids], vals
                        )

        for b in range(2):
            pltpu.sync_copy(idx_hbm.at[pl.ds(idx_off(b), BLK)], idx_v.at[b])
            gather_start(b)

        def super_body(s, carry):
            for j in range(2):
                t = s * 2 + j
                b = j

                @pl.when(s > 0)
                def _():
                    write_wait(b)

                gather_wait(b)

                @pl.when(s < (H // 2) - 1)
                def _():
                    idx_start(t + 2, b)

                transpose(b)
                write_start(t, b)

                @pl.when(s < (H // 2) - 1)
                def _():
                    idx_wait(b)
                    gather_start(b)

            return carry

        lax.fori_loop(0, H // 2, super_body, 0)

        for b in range(2):
            write_wait(b)

    return lookup


def kernel(input_seq, embedding_table):
    Bt, H = input_seq.shape
    V, D = embedding_table.shape
    Hp = (H + SUBLANE - 1) // SUBLANE * SUBLANE
    idxT = jnp.pad(input_seq.T.astype(jnp.int32), ((0, Hp - H), (0, 0)))
    outT = _make_lookup(V, D, Bt, H)(embedding_table, idxT.reshape(Hp * Bt))
    return outT.transpose(2, 0, 1)


# speedup vs baseline: 4.3835x; 1.0025x over previous
import functools

import jax
import jax.numpy as jnp
from jax import lax
from jax.experimental import pallas as pl
from jax.experimental.pallas import tpu as pltpu
from jax.experimental.pallas import tpu_sc as plsc

NUM_CORES = 2
NUM_SUBCORES = 16
NUM_WORKERS = NUM_CORES * NUM_SUBCORES
SUBLANE = 8
LANES = 16


@functools.lru_cache(maxsize=None)
def _make_lookup(V, D, Bt, H):
    BLK = Bt // NUM_WORKERS
    assert Bt % NUM_WORKERS == 0 and BLK % LANES == 0 and D % LANES == 0
    assert H % 2 == 0
    mesh = plsc.VectorSubcoreMesh(core_axis_name="c", subcore_axis_name="s")

    @functools.partial(
        pl.kernel,
        mesh=mesh,
        out_type=jax.ShapeDtypeStruct((H * D, Bt), jnp.float32),
        scratch_types=[
            pltpu.VMEM((2, BLK), jnp.int32),
            pltpu.VMEM((2, BLK, D), jnp.float32),
            pltpu.VMEM((2, D, BLK), jnp.float32),
            pltpu.SemaphoreType.DMA((2,)),
            pltpu.SemaphoreType.DMA((2,)),
            pltpu.SemaphoreType.DMA((2,)),
        ],
        compiler_params=pltpu.CompilerParams(
            use_tc_tiling_on_sc=False, needs_layout_passes=False
        ),
    )
    def lookup(tab_hbm, idx_hbm, out_hbm, idx_v, rows_v, tr_v, sem_i, sem_g, sem_o):
        wid = lax.axis_index("s") * NUM_CORES + lax.axis_index("c")
        col0 = wid * BLK
        iota = lax.iota(jnp.int32, LANES)
        rots = [jnp.bitwise_and(iota + k, LANES - 1) for k in range(LANES)]

        def idx_off(t):
            return t * Bt + col0

        def idx_start(t, b):
            pltpu.async_copy(
                idx_hbm.at[pl.ds(idx_off(t), BLK)], idx_v.at[b], sem_i.at[b]
            )

        def idx_wait(b):
            pltpu.make_async_copy(
                idx_hbm.at[pl.ds(0, BLK)], idx_v.at[b], sem_i.at[b]
            ).wait()

        def gather_start(b):
            pltpu.async_copy(tab_hbm.at[idx_v.at[b]], rows_v.at[b], sem_g.at[b])

        def gather_wait(b):
            pltpu.make_async_copy(
                tab_hbm.at[idx_v.at[b]], rows_v.at[b], sem_g.at[b]
            ).wait()

        def write_start(t, b):
            pltpu.async_copy(
                tr_v.at[b],
                out_hbm.at[pl.ds(t * D, D), pl.ds(col0, BLK)],
                sem_o.at[b],
            )

        def write_wait(b):
            pltpu.make_async_copy(
                tr_v.at[b], out_hbm.at[pl.ds(0, D), pl.ds(0, BLK)], sem_o.at[b]
            ).wait()

        def transpose(b):
            @plsc.parallel_loop(0, BLK // LANES, 1, unroll=2)
            def r_body(r0):
                row_ids = r0 * LANES + iota
                for c0 in range(D // LANES):
                    for k in range(LANES):
                        col_ids = c0 * LANES + rots[k]
                        vals = plsc.load_gather(
                            rows_v.at[b], [row_ids, col_ids]
                        )
                        plsc.store_scatter(
                            tr_v.at[b], [col_ids, row_ids], vals
                        )

        for b in range(2):
            pltpu.sync_copy(idx_hbm.at[pl.ds(idx_off(b), BLK)], idx_v.at[b])
            gather_start(b)

        def super_body(s, carry):
            for j in range(2):
                t = s * 2 + j
                b = j

                @pl.when(s > 0)
                def _():
                    write_wait(b)

                gather_wait(b)

                @pl.when(s < (H // 2) - 1)
                def _():
                    idx_start(t + 2, b)

                transpose(b)
                write_start(t, b)

                @pl.when(s < (H // 2) - 1)
                def _():
                    idx_wait(b)
                    gather_start(b)

            return carry

        lax.fori_loop(0, H // 2, super_body, 0)

        for b in range(2):
            write_wait(b)

    return lookup


def kernel(input_seq, embedding_table):
    Bt, H = input_seq.shape
    V, D = embedding_table.shape
    Hp = (H + SUBLANE - 1) // SUBLANE * SUBLANE
    idxT = jnp.pad(input_seq.T.astype(jnp.int32), ((0, Hp - H), (0, 0)))
    outT = _make_lookup(V, D, Bt, H)(embedding_table, idxT.reshape(Hp * Bt))
    return outT.reshape(H, D, Bt).transpose(2, 0, 1)


# speedup vs baseline: 4.3999x; 1.0037x over previous
import functools

import jax
import jax.numpy as jnp
from jax import lax
from jax.experimental import pallas as pl
from jax.experimental.pallas import tpu as pltpu
from jax.experimental.pallas import tpu_sc as plsc

NUM_CORES = 2
NUM_SUBCORES = 16
NUM_WORKERS = NUM_CORES * NUM_SUBCORES
SUBLANE = 8
LANES = 16


@functools.lru_cache(maxsize=None)
def _make_lookup(V, D, Bt, H):
    BLK = Bt // NUM_WORKERS
    assert Bt % NUM_WORKERS == 0 and BLK % LANES == 0 and D % LANES == 0
    assert H % 2 == 0
    mesh = plsc.VectorSubcoreMesh(core_axis_name="c", subcore_axis_name="s")

    @functools.partial(
        pl.kernel,
        mesh=mesh,
        out_type=jax.ShapeDtypeStruct((H * D, Bt), jnp.float32),
        scratch_types=[
            pltpu.VMEM((2, BLK), jnp.int32),
            pltpu.VMEM((2, BLK, D), jnp.float32),
            pltpu.VMEM((2, D, BLK), jnp.float32),
            pltpu.SemaphoreType.DMA((2,)),
            pltpu.SemaphoreType.DMA((2,)),
            pltpu.SemaphoreType.DMA((2,)),
        ],
        compiler_params=pltpu.CompilerParams(
            use_tc_tiling_on_sc=False, needs_layout_passes=False
        ),
    )
    def lookup(tab_hbm, idx_hbm, out_hbm, idx_v, rows_v, tr_v, sem_i, sem_g, sem_o):
        wid = lax.axis_index("s") * NUM_CORES + lax.axis_index("c")
        col0 = wid * BLK
        iota = lax.iota(jnp.int32, LANES)
        rots = [jnp.bitwise_and(iota + k, LANES - 1) for k in range(LANES)]

        def idx_off(t):
            return t * Bt + col0

        def idx_start(t, b):
            pltpu.async_copy(
                idx_hbm.at[pl.ds(idx_off(t), BLK)], idx_v.at[b], sem_i.at[b]
            )

        def idx_wait(b):
            pltpu.make_async_copy(
                idx_hbm.at[pl.ds(0, BLK)], idx_v.at[b], sem_i.at[b]
            ).wait()

        def gather_start(b):
            pltpu.async_copy(tab_hbm.at[idx_v.at[b]], rows_v.at[b], sem_g.at[b])

        def gather_wait(b):
            pltpu.make_async_copy(
                tab_hbm.at[idx_v.at[b]], rows_v.at[b], sem_g.at[b]
            ).wait()

        def write_start(t, b):
            pltpu.async_copy(
                tr_v.at[b],
                out_hbm.at[pl.ds(t * D, D), pl.ds(col0, BLK)],
                sem_o.at[b],
            )

        def write_wait(b):
            pltpu.make_async_copy(
                tr_v.at[b], out_hbm.at[pl.ds(0, D), pl.ds(0, BLK)], sem_o.at[b]
            ).wait()

        def transpose(b):
            @plsc.parallel_loop(0, BLK // LANES, 1, unroll=4)
            def r_body(r0):
                row_ids = r0 * LANES + iota
                for c0 in range(D // LANES):
                    for k in range(LANES):
                        col_ids = c0 * LANES + rots[k]
                        vals = plsc.load_gather(
                            rows_v.at[b], [row_ids, col_ids]
                        )
                        plsc.store_scatter(
                            tr_v.at[b], [col_ids, row_ids], vals
                        )

        for b in range(2):
            pltpu.sync_copy(idx_hbm.at[pl.ds(idx_off(b), BLK)], idx_v.at[b])
            gather_start(b)

        def super_body(s, carry):
            for j in range(2):
                t = s * 2 + j
                b = j

                @pl.when(s > 0)
                def _():
                    write_wait(b)

                gather_wait(b)

                @pl.when(s < (H // 2) - 1)
                def _():
                    idx_start(t + 2, b)

                transpose(b)
                write_start(t, b)

                @pl.when(s < (H // 2) - 1)
                def _():
                    idx_wait(b)
                    gather_start(b)

            return carry

        lax.fori_loop(0, H // 2, super_body, 0)

        for b in range(2):
            write_wait(b)

    return lookup


def kernel(input_seq, embedding_table):
    Bt, H = input_seq.shape
    V, D = embedding_table.shape
    Hp = (H + SUBLANE - 1) // SUBLANE * SUBLANE
    idxT = jnp.pad(input_seq.T.astype(jnp.int32), ((0, Hp - H), (0, 0)))
    outT = _make_lookup(V, D, Bt, H)(embedding_table, idxT.reshape(Hp * Bt))
    return outT.reshape(H, D, Bt).transpose(2, 0, 1)


# speedup vs baseline: 5.0467x; 1.1470x over previous
import functools

import jax
import jax.numpy as jnp
from jax import lax
from jax.experimental import pallas as pl
from jax.experimental.pallas import tpu as pltpu
from jax.experimental.pallas import tpu_sc as plsc

NUM_CORES = 2
NUM_SUBCORES = 16
NUM_WORKERS = NUM_CORES * NUM_SUBCORES
SUBLANE = 8
LANES = 16


@functools.lru_cache(maxsize=None)
def _make_pack(V, D):
    CH = 512
    n_full = V // CH
    tail = V - n_full * CH
    n_rr = n_full // NUM_WORKERS
    n_extra = n_full - n_rr * NUM_WORKERS
    G = 128 // D
    mesh = plsc.VectorSubcoreMesh(core_axis_name="c", subcore_axis_name="s")

    @functools.partial(
        pl.kernel,
        mesh=mesh,
        out_type=jax.ShapeDtypeStruct((V // G, G * D), jnp.float32),
        scratch_types=[
            pltpu.VMEM((D, CH), jnp.float32),
            pltpu.VMEM((CH // G, G * D), jnp.float32),
            pltpu.VMEM((max(tail, G), D), jnp.float32),
        ],
        compiler_params=pltpu.CompilerParams(needs_layout_passes=False),
    )
    def pack(tabT_hbm, tail_hbm, out_hbm, in_v, pk_v, tv):
        wid = lax.axis_index("s") * NUM_CORES + lax.axis_index("c")
        iota = lax.iota(jnp.int32, LANES)
        rots = [jnp.bitwise_and(iota + k, LANES - 1) for k in range(LANES)]
        iota_div = lax.shift_right_logical(iota, 2)
        lane_c = [
            [
                (jnp.bitwise_and(iota, G - 1)) * D + d0 * LANES + rots[k]
                for k in range(LANES)
            ]
            for d0 in range(D // LANES)
        ]

        def do_chunk(c):
            col0 = pl.multiple_of(c * CH, CH)
            pltpu.sync_copy(tabT_hbm.at[:, pl.ds(col0, CH)], in_v)

            def jb_body(jb, carry):
                row_ids = jb * G + iota_div
                for d0 in range(D // LANES):
                    for k in range(LANES):
                        j_ids = jb * LANES + iota
                        vals = plsc.load_gather(in_v, [d0 * LANES + rots[k], j_ids])
                        plsc.store_scatter(pk_v, [row_ids, lane_c[d0][k]], vals)
                return carry

            lax.fori_loop(0, CH // LANES, jb_body, 0)
            row0 = pl.multiple_of(c * (CH // G), CH // G)
            pltpu.sync_copy(pk_v, out_hbm.at[pl.ds(row0, CH // G), :])

        def s_body(s, carry):
            do_chunk(wid + NUM_WORKERS * s)
            return carry

        lax.fori_loop(0, n_rr, s_body, 0)

        for e in range(n_extra):
            @pl.when(wid == e)
            def _():
                do_chunk(n_rr * NUM_WORKERS + e)

        if tail:
            @pl.when(wid == n_extra)
            def _():
                pltpu.sync_copy(tail_hbm, tv)

                def v_body(v, carry):
                    row = lax.shift_right_logical(v, 2)
                    lane0 = pl.multiple_of(jnp.bitwise_and(v, G - 1) * D, D)
                    for h in range(D // LANES):
                        pk_v[row, pl.ds(lane0 + h * LANES, LANES)] = tv[
                            v, pl.ds(h * LANES, LANES)
                        ]
                    return carry

                lax.fori_loop(0, tail, v_body, 0)
                pltpu.sync_copy(
                    pk_v.at[pl.ds(0, tail // G), :],
                    out_hbm.at[pl.ds((V - tail) // G, tail // G), :],
                )

    return pack


@functools.lru_cache(maxsize=None)
def _make_lookup(V, D, Bt, H):
    BLK = Bt // NUM_WORKERS
    assert Bt % NUM_WORKERS == 0 and BLK % LANES == 0 and D % LANES == 0
    assert H % 2 == 0
    mesh = plsc.VectorSubcoreMesh(core_axis_name="c", subcore_axis_name="s")

    @functools.partial(
        pl.kernel,
        mesh=mesh,
        out_type=jax.ShapeDtypeStruct((H * D, Bt), jnp.float32),
        scratch_types=[
            pltpu.VMEM((2, BLK), jnp.int32),
            pltpu.VMEM((2, BLK, D), jnp.float32),
            pltpu.VMEM((2, D, BLK), jnp.float32),
            pltpu.SemaphoreType.DMA((2,)),
            pltpu.SemaphoreType.DMA((2,)),
            pltpu.SemaphoreType.DMA((2,)),
        ],
        compiler_params=pltpu.CompilerParams(
            use_tc_tiling_on_sc=False, needs_layout_passes=False
        ),
    )
    def lookup(tab_hbm, idx_hbm, out_hbm, idx_v, rows_v, tr_v, sem_i, sem_g, sem_o):
        wid = lax.axis_index("s") * NUM_CORES + lax.axis_index("c")
        col0 = wid * BLK
        iota = lax.iota(jnp.int32, LANES)
        rots = [jnp.bitwise_and(iota + k, LANES - 1) for k in range(LANES)]

        def idx_off(t):
            return t * Bt + col0

        def idx_start(t, b):
            pltpu.async_copy(
                idx_hbm.at[pl.ds(idx_off(t), BLK)], idx_v.at[b], sem_i.at[b]
            )

        def idx_wait(b):
            pltpu.make_async_copy(
                idx_hbm.at[pl.ds(0, BLK)], idx_v.at[b], sem_i.at[b]
            ).wait()

        def gather_start(b):
            pltpu.async_copy(tab_hbm.at[idx_v.at[b]], rows_v.at[b], sem_g.at[b])

        def gather_wait(b):
            pltpu.make_async_copy(
                tab_hbm.at[idx_v.at[b]], rows_v.at[b], sem_g.at[b]
            ).wait()

        def write_start(t, b):
            pltpu.async_copy(
                tr_v.at[b],
                out_hbm.at[pl.ds(t * D, D), pl.ds(col0, BLK)],
                sem_o.at[b],
            )

        def write_wait(b):
            pltpu.make_async_copy(
                tr_v.at[b], out_hbm.at[pl.ds(0, D), pl.ds(0, BLK)], sem_o.at[b]
            ).wait()

        def transpose(b):
            @plsc.parallel_loop(0, BLK // LANES, 1, unroll=4)
            def r_body(r0):
                row_ids = r0 * LANES + iota
                for c0 in range(D // LANES):
                    for k in range(LANES):
                        col_ids = c0 * LANES + rots[k]
                        vals = plsc.load_gather(
                            rows_v.at[b], [row_ids, col_ids]
                        )
                        plsc.store_scatter(
                            tr_v.at[b], [col_ids, row_ids], vals
                        )

        for b in range(2):
            pltpu.sync_copy(idx_hbm.at[pl.ds(idx_off(b), BLK)], idx_v.at[b])
            gather_start(b)

        def super_body(s, carry):
            for j in range(2):
                t = s * 2 + j
                b = j

                @pl.when(s > 0)
                def _():
                    write_wait(b)

                gather_wait(b)

                @pl.when(s < (H // 2) - 1)
                def _():
                    idx_start(t + 2, b)

                transpose(b)
                write_start(t, b)

                @pl.when(s < (H // 2) - 1)
                def _():
                    idx_wait(b)
                    gather_start(b)

            return carry

        lax.fori_loop(0, H // 2, super_body, 0)

        for b in range(2):
            write_wait(b)

    return lookup


def kernel(input_seq, embedding_table):
    Bt, H = input_seq.shape
    V, D = embedding_table.shape
    Hp = (H + SUBLANE - 1) // SUBLANE * SUBLANE
    idxT = jnp.pad(input_seq.T.astype(jnp.int32), ((0, Hp - H), (0, 0)))
    tab_lin = _make_pack(V, D)(
        embedding_table.T, embedding_table[V - V % 512 :, :]
    ).reshape(V, D)
    outT = _make_lookup(V, D, Bt, H)(tab_lin, idxT.reshape(Hp * Bt))
    return outT.reshape(H, D, Bt).transpose(2, 0, 1)


# speedup vs baseline: 6.3959x; 1.2673x over previous
import functools

import jax
import jax.numpy as jnp
from jax import lax
from jax.experimental import pallas as pl
from jax.experimental.pallas import tpu as pltpu
from jax.experimental.pallas import tpu_sc as plsc

NUM_CORES = 2
NUM_SUBCORES = 16
NUM_WORKERS = NUM_CORES * NUM_SUBCORES
SUBLANE = 8
LANES = 16


@functools.lru_cache(maxsize=None)
def _make_pack(V, D):
    CH = 512
    n_full = V // CH
    tail = V - n_full * CH
    n_rr = n_full // NUM_WORKERS
    n_extra = n_full - n_rr * NUM_WORKERS
    G = 128 // D
    mesh = plsc.VectorSubcoreMesh(core_axis_name="c", subcore_axis_name="s")

    @functools.partial(
        pl.kernel,
        mesh=mesh,
        out_type=jax.ShapeDtypeStruct((V // G, G * D), jnp.float32),
        scratch_types=[
            pltpu.VMEM((2, D, CH), jnp.float32),
            pltpu.VMEM((2, CH // G, G * D), jnp.float32),
            pltpu.VMEM((max(tail, G), D), jnp.float32),
            pltpu.SemaphoreType.DMA((2,)),
            pltpu.SemaphoreType.DMA((2,)),
        ],
        compiler_params=pltpu.CompilerParams(needs_layout_passes=False),
    )
    def pack(tabT_hbm, tail_hbm, out_hbm, in_v, pk_v, tv, sem_r, sem_w):
        wid = lax.axis_index("s") * NUM_CORES + lax.axis_index("c")
        iota = lax.iota(jnp.int32, LANES)
        rots = [jnp.bitwise_and(iota + k, LANES - 1) for k in range(LANES)]
        iota_div = lax.shift_right_logical(iota, 2)
        lane_c = [
            [
                (jnp.bitwise_and(iota, G - 1)) * D + d0 * LANES + rots[k]
                for k in range(LANES)
            ]
            for d0 in range(D // LANES)
        ]

        def chunk_of(t):
            return wid + NUM_WORKERS * t

        def rd_start(c, b):
            col0 = pl.multiple_of(c * CH, CH)
            pltpu.async_copy(
                tabT_hbm.at[:, pl.ds(col0, CH)], in_v.at[b], sem_r.at[b]
            )

        def rd_wait(b):
            pltpu.make_async_copy(
                tabT_hbm.at[:, pl.ds(0, CH)], in_v.at[b], sem_r.at[b]
            ).wait()

        def wr_start(c, b):
            row0 = pl.multiple_of(c * (CH // G), CH // G)
            pltpu.async_copy(
                pk_v.at[b], out_hbm.at[pl.ds(row0, CH // G), :], sem_w.at[b]
            )

        def wr_wait(b):
            pltpu.make_async_copy(
                pk_v.at[b], out_hbm.at[pl.ds(0, CH // G), :], sem_w.at[b]
            ).wait()

        def compute(b):
            def jb_body(jb, carry):
                row_ids = jb * G + iota_div
                for d0 in range(D // LANES):
                    for k in range(LANES):
                        j_ids = jb * LANES + iota
                        vals = plsc.load_gather(
                            in_v.at[b], [d0 * LANES + rots[k], j_ids]
                        )
                        plsc.store_scatter(
                            pk_v.at[b], [row_ids, lane_c[d0][k]], vals
                        )
                return carry

            lax.fori_loop(0, CH // LANES, jb_body, 0)

        def do_chunk_sync(c):
            col0 = pl.multiple_of(c * CH, CH)
            pltpu.sync_copy(tabT_hbm.at[:, pl.ds(col0, CH)], in_v.at[0])
            compute(0)
            row0 = pl.multiple_of(c * (CH // G), CH // G)
            pltpu.sync_copy(pk_v.at[0], out_hbm.at[pl.ds(row0, CH // G), :])

        n_loop = n_rr - (n_rr % 2)
        rd_start(chunk_of(0), 0)

        def s_body(s, carry):
            for j in range(2):
                t = s * 2 + j
                b = j
                rd_wait(b)

                @pl.when(t + 1 < n_loop)
                def _():
                    rd_start(chunk_of(t + 1), 1 - b)

                @pl.when(s > 0)
                def _():
                    wr_wait(b)

                compute(b)
                wr_start(chunk_of(t), b)
            return carry

        lax.fori_loop(0, n_loop // 2, s_body, 0)

        for b in range(2):
            wr_wait(b)

        if n_rr % 2:
            do_chunk_sync(chunk_of(n_rr - 1))

        for e in range(n_extra):
            @pl.when(wid == e)
            def _():
                do_chunk_sync(n_rr * NUM_WORKERS + e)

        if tail:
            @pl.when(wid == n_extra)
            def _():
                pltpu.sync_copy(tail_hbm, tv)

                def v_body(v, carry):
                    row = lax.shift_right_logical(v, 2)
                    lane0 = pl.multiple_of(jnp.bitwise_and(v, G - 1) * D, D)
                    for h in range(D // LANES):
                        pk_v[0, row, pl.ds(lane0 + h * LANES, LANES)] = tv[
                            v, pl.ds(h * LANES, LANES)
                        ]
                    return carry

                lax.fori_loop(0, tail, v_body, 0)
                pltpu.sync_copy(
                    pk_v.at[0, pl.ds(0, tail // G), :],
                    out_hbm.at[pl.ds((V - tail) // G, tail // G), :],
                )

    return pack


@functools.lru_cache(maxsize=None)
def _make_lookup(V, D, Bt, H):
    BLK = Bt // NUM_WORKERS
    assert Bt % NUM_WORKERS == 0 and BLK % LANES == 0 and D % LANES == 0
    assert H % 2 == 0
    mesh = plsc.VectorSubcoreMesh(core_axis_name="c", subcore_axis_name="s")

    @functools.partial(
        pl.kernel,
        mesh=mesh,
        out_type=jax.ShapeDtypeStruct((H * D, Bt), jnp.float32),
        scratch_types=[
            pltpu.VMEM((2, BLK), jnp.int32),
            pltpu.VMEM((2, BLK, D), jnp.float32),
            pltpu.VMEM((2, D, BLK), jnp.float32),
            pltpu.SemaphoreType.DMA((2,)),
            pltpu.SemaphoreType.DMA((2,)),
            pltpu.SemaphoreType.DMA((2,)),
        ],
        compiler_params=pltpu.CompilerParams(
            use_tc_tiling_on_sc=False, needs_layout_passes=False
        ),
    )
    def lookup(tab_hbm, idx_hbm, out_hbm, idx_v, rows_v, tr_v, sem_i, sem_g, sem_o):
        wid = lax.axis_index("s") * NUM_CORES + lax.axis_index("c")
        col0 = wid * BLK
        iota = lax.iota(jnp.int32, LANES)
        rots = [jnp.bitwise_and(iota + k, LANES - 1) for k in range(LANES)]

        def idx_off(t):
            return t * Bt + col0

        def idx_start(t, b):
            pltpu.async_copy(
                idx_hbm.at[pl.ds(idx_off(t), BLK)], idx_v.at[b], sem_i.at[b]
            )

        def idx_wait(b):
            pltpu.make_async_copy(
                idx_hbm.at[pl.ds(0, BLK)], idx_v.at[b], sem_i.at[b]
            ).wait()

        def gather_start(b):
            pltpu.async_copy(tab_hbm.at[idx_v.at[b]], rows_v.at[b], sem_g.at[b])

        def gather_wait(b):
            pltpu.make_async_copy(
                tab_hbm.at[idx_v.at[b]], rows_v.at[b], sem_g.at[b]
            ).wait()

        def write_start(t, b):
            pltpu.async_copy(
                tr_v.at[b],
                out_hbm.at[pl.ds(t * D, D), pl.ds(col0, BLK)],
                sem_o.at[b],
            )

        def write_wait(b):
            pltpu.make_async_copy(
                tr_v.at[b], out_hbm.at[pl.ds(0, D), pl.ds(0, BLK)], sem_o.at[b]
            ).wait()

        def transpose(b):
            @plsc.parallel_loop(0, BLK // LANES, 1, unroll=4)
            def r_body(r0):
                row_ids = r0 * LANES + iota
                for c0 in range(D // LANES):
                    for k in range(LANES):
                        col_ids = c0 * LANES + rots[k]
                        vals = plsc.load_gather(
                            rows_v.at[b], [row_ids, col_ids]
                        )
                        plsc.store_scatter(
                            tr_v.at[b], [col_ids, row_ids], vals
                        )

        for b in range(2):
            pltpu.sync_copy(idx_hbm.at[pl.ds(idx_off(b), BLK)], idx_v.at[b])
            gather_start(b)

        def super_body(s, carry):
            for j in range(2):
                t = s * 2 + j
                b = j

                @pl.when(s > 0)
                def _():
                    write_wait(b)

                gather_wait(b)

                @pl.when(s < (H // 2) - 1)
                def _():
                    idx_start(t + 2, b)

                transpose(b)
                write_start(t, b)

                @pl.when(s < (H // 2) - 1)
                def _():
                    idx_wait(b)
                    gather_start(b)

            return carry

        lax.fori_loop(0, H // 2, super_body, 0)

        for b in range(2):
            write_wait(b)

    return lookup


def kernel(input_seq, embedding_table):
    Bt, H = input_seq.shape
    V, D = embedding_table.shape
    Hp = (H + SUBLANE - 1) // SUBLANE * SUBLANE
    idxT = jnp.pad(input_seq.T.astype(jnp.int32), ((0, Hp - H), (0, 0)))
    tab_lin = _make_pack(V, D)(
        embedding_table.T, embedding_table[V - V % 512 :, :]
    ).reshape(V, D)
    outT = _make_lookup(V, D, Bt, H)(tab_lin, idxT.reshape(Hp * Bt))
    return outT.reshape(H, D, Bt).transpose(2, 0, 1)
